# baseline (speedup 1.0000x reference)
import functools

import jax
import jax.numpy as jnp
from jax import lax
from jax.experimental import pallas as pl
from jax.experimental.pallas import tpu as pltpu
from jax.experimental.pallas import tpu_sc as plsc

N = 10000
K = 32
C = 128
R = 0.1
NV = C // 16

N_PAD = 10240
NC = 2
NS = 16
NW = NC * NS
PW0 = 208
PW1 = 432
A_PTS = NS * PW0
PWMAX = PW1
G = 4

_SIGMA = R * 0.3
_WSCALE = -1.0 / (2.0 * _SIGMA * _SIGMA + 1e-9)

_BLK = 1280


def _dense_body(x_ref, w_ref, b_ref, g_ref, bt_ref, xpad_ref, o_ref):
    i = pl.program_id(0)
    h = lax.dot_general(x_ref[...], w_ref[...], (((1,), (1,)), ((), ())),
                        preferred_element_type=jnp.float32)
    h = h + b_ref[...]
    mu = jnp.mean(h, axis=-1, keepdims=True)
    var = jnp.mean((h - mu) ** 2, axis=-1, keepdims=True)
    h = (h - mu) * lax.rsqrt(var + 1e-5) * g_ref[...] + bt_ref[...]
    h = jnp.where(h >= 0, h, 0.1 * h)
    rows = i * _BLK + lax.broadcasted_iota(jnp.int32, (_BLK, 1), 0)
    o_ref[...] = jnp.where(rows >= N, xpad_ref[...], h)


def _dense(x_padded, W, b, gamma, beta, x_pad):
    return pl.pallas_call(
        _dense_body,
        grid=(N_PAD // _BLK,),
        in_specs=[
            pl.BlockSpec((_BLK, C), lambda i: (i, 0)),
            pl.BlockSpec((C, C), lambda i: (0, 0)),
            pl.BlockSpec((1, C), lambda i: (0, 0)),
            pl.BlockSpec((1, C), lambda i: (0, 0)),
            pl.BlockSpec((1, C), lambda i: (0, 0)),
            pl.BlockSpec((1, C), lambda i: (0, 0)),
        ],
        out_specs=pl.BlockSpec((_BLK, C), lambda i: (i, 0)),
        out_shape=jax.ShapeDtypeStruct((N_PAD, C), jnp.float32),
    )(x_padded, W, b, gamma, beta, x_pad)


_mesh = plsc.VectorSubcoreMesh(core_axis_name="c", subcore_axis_name="s")


@functools.partial(
    pl.kernel,
    out_type=jax.ShapeDtypeStruct((N_PAD, C), jnp.float32),
    mesh=_mesh,
    scratch_types=[
        pltpu.VMEM((N_PAD,), jnp.float32),
        pltpu.VMEM((N_PAD,), jnp.float32),
        pltpu.VMEM((N_PAD,), jnp.float32),
        pltpu.VMEM((PWMAX * K,), jnp.int32),
        pltpu.VMEM((PWMAX * K,), jnp.int32),
        pltpu.VMEM((PWMAX,), jnp.float32),
        pltpu.VMEM((PWMAX,), jnp.float32),
        pltpu.VMEM((PWMAX,), jnp.float32),
        pltpu.VMEM((PWMAX * K,), jnp.float32),
        pltpu.VMEM((2, G * K, C), jnp.float32),
        pltpu.VMEM((G, C), jnp.float32),
        pltpu.SemaphoreType.DMA,
        pltpu.SemaphoreType.DMA,
    ],
    compiler_params=pltpu.CompilerParams(needs_layout_passes=False),
)
def _sc_pool(table_hbm, idx_hbm, idxt_hbm, px_hbm, py_hbm, pz_hbm,
             qx_hbm, qy_hbm, qz_hbm, out_hbm,
             px_v, py_v, pz_v, idx_v, idxt_v, qx_v, qy_v, qz_v,
             w_v, rows_v, out_v, sem0, sem1):
    cid = lax.axis_index("c")
    sid = lax.axis_index("s")
    is_a = cid == 0
    pw = jnp.where(is_a, PW0, PW1)
    pbase = jnp.where(is_a, sid * PW0, A_PTS + sid * PW1)
    ss = jnp.where(is_a, PW0 // 16, PW1 // 16)
    ngrp = jnp.where(is_a, PW0 // G, PW1 // G)

    pltpu.sync_copy(px_hbm, px_v)
    pltpu.sync_copy(py_hbm, py_v)
    pltpu.sync_copy(pz_hbm, pz_v)
    pltpu.sync_copy(idx_hbm.at[pl.ds(pbase * K, PWMAX * K)], idx_v)
    pltpu.sync_copy(idxt_hbm.at[pl.ds(pbase * K, PWMAX * K)], idxt_v)
    pltpu.sync_copy(qx_hbm.at[pl.ds(pbase, PWMAX)], qx_v)
    pltpu.sync_copy(qy_hbm.at[pl.ds(pbase, PWMAX)], qy_v)
    pltpu.sync_copy(qz_hbm.at[pl.ds(pbase, PWMAX)], qz_v)

    def _start(g, buf, sem):
        idx_slice = idx_v.at[pl.ds(g * (G * K), G * K)]
        pltpu.async_copy(table_hbm.at[idx_slice], rows_v.at[buf], sem)

    def _wait(g, buf, sem):
        pltpu.make_async_copy(
            table_hbm.at[idx_v.at[pl.ds(g * (G * K), G * K)]],
            rows_v.at[buf], sem).wait()

    _start(0, 0, sem0)
    _start(1, 1, sem1)

    lanes_k = lax.iota(jnp.int32, 16) * K

    def _weights_body(s, carry):
        soff = pl.multiple_of(s * 16, 16)
        qx = qx_v[pl.ds(soff, 16)]
        qy = qy_v[pl.ds(soff, 16)]
        qz = qz_v[pl.ds(soff, 16)]
        den = jnp.zeros((16,), jnp.float32)
        wbase = s * (16 * K)
        for j in range(K):
            ivec = idxt_v[pl.ds(pl.multiple_of(j * pw + s * 16, 16), 16)]
            dx = qx - plsc.load_gather(px_v, [ivec])
            dy = qy - plsc.load_gather(py_v, [ivec])
            dz = qz - plsc.load_gather(pz_v, [ivec])
            d = dx * dx + dy * dy + dz * dz
            w = jnp.maximum(jnp.exp(d * _WSCALE), 0.001)
            den = den + w
            plsc.store_scatter(w_v, [wbase + lanes_k + j], w)
        inv = 1.0 / den
        for i in range(16):
            woff = pl.multiple_of(wbase + i * K, 16)
            w_v[pl.ds(woff, 16)] = w_v[pl.ds(woff, 16)] * inv[i]
            w_v[pl.ds(woff + 16, 16)] = w_v[pl.ds(woff + 16, 16)] * inv[i]
        return carry

    lax.fori_loop(0, ss, _weights_body, 0)

    def _group(g, buf):
        for i in range(G):
            woff = pl.multiple_of((g * G + i) * K, 16)
            wv0 = w_v[pl.ds(woff, 16)]
            wv1 = w_v[pl.ds(woff + 16, 16)]
            acc = [jnp.zeros((16,), jnp.float32) for _ in range(NV)]
            for j in range(K):
                wj = wv0[j] if j < 16 else wv1[j - 16]
                row = i * K + j
                for c in range(NV):
                    acc[c] = acc[c] + wj * rows_v[buf, row, pl.ds(c * 16, 16)]
            for c in range(NV):
                out_v[i, pl.ds(c * 16, 16)] = acc[c]
        pltpu.sync_copy(out_v, out_hbm.at[pl.ds(pbase + g * G, G)])

    def _pool_body(gp, carry):
        g = 2 * gp
        _wait(g, 0, sem0)
        _group(g, 0)

        @pl.when(g + 2 < ngrp)
        def _():
            _start(g + 2, 0, sem0)

        _wait(g + 1, 1, sem1)
        _group(g + 1, 1)

        @pl.when(g + 3 < ngrp)
        def _():
            _start(g + 3, 1, sem1)

        return carry

    lax.fori_loop(0, ngrp // 2, _pool_body, 0)


def kernel(x, x_pad, idx, neighbor_pts, query_pts, W, b, gamma, beta):
    x_padded = jnp.zeros((N_PAD, C), jnp.float32).at[:N].set(x)
    table = _dense(x_padded, W, b.reshape(1, C), gamma.reshape(1, C),
                   beta.reshape(1, C), x_pad)

    pts = jnp.full((N_PAD, 3), 1e6, jnp.float32).at[:N].set(neighbor_pts)
    q = jnp.zeros((N_PAD, 3), jnp.float32).at[:N].set(query_pts)
    idx_pad = jnp.zeros((N_PAD, K), jnp.int32).at[:N].set(idx.astype(jnp.int32))
    idx_flat = idx_pad.reshape(-1)
    idxt_a = idx_pad[:A_PTS].reshape(NS, PW0, K).transpose(0, 2, 1)
    idxt_b = idx_pad[A_PTS:].reshape(NS, PW1, K).transpose(0, 2, 1)
    idxt_flat = jnp.concatenate([idxt_a.reshape(-1), idxt_b.reshape(-1)])

    out = _sc_pool(table, idx_flat, idxt_flat,
                   pts[:, 0].copy(), pts[:, 1].copy(), pts[:, 2].copy(),
                   q[:, 0].copy(), q[:, 1].copy(), q[:, 2].copy())
    return out[:N]


# speedup vs baseline: 1.0631x; 1.0631x over previous
import functools

import jax
import jax.numpy as jnp
from jax import lax
from jax.experimental import pallas as pl
from jax.experimental.pallas import tpu as pltpu
from jax.experimental.pallas import tpu_sc as plsc

N = 10000
K = 32
C = 128
R = 0.1
NV = C // 16

N_PAD = 10240
NC = 2
NS = 16
NW = NC * NS
PW0 = 208
PW1 = 432
A_PTS = NS * PW0
PWMAX = PW1
G = 4

_SIGMA = R * 0.3
_WSCALE = -1.0 / (2.0 * _SIGMA * _SIGMA + 1e-9)

_BLK = 1280


def _dense_body(x_ref, w_ref, b_ref, g_ref, bt_ref, xpad_ref, o_ref):
    i = pl.program_id(0)
    h = lax.dot_general(x_ref[...], w_ref[...], (((1,), (1,)), ((), ())),
                        preferred_element_type=jnp.float32)
    h = h + b_ref[...]
    mu = jnp.mean(h, axis=-1, keepdims=True)
    var = jnp.mean((h - mu) ** 2, axis=-1, keepdims=True)
    h = (h - mu) * lax.rsqrt(var + 1e-5) * g_ref[...] + bt_ref[...]
    h = jnp.where(h >= 0, h, 0.1 * h)
    rows = i * _BLK + lax.broadcasted_iota(jnp.int32, (_BLK, 1), 0)
    o_ref[...] = jnp.where(rows >= N, xpad_ref[...], h)


def _dense(x_padded, W, b, gamma, beta, x_pad):
    return pl.pallas_call(
        _dense_body,
        grid=(N_PAD // _BLK,),
        in_specs=[
            pl.BlockSpec((_BLK, C), lambda i: (i, 0)),
            pl.BlockSpec((C, C), lambda i: (0, 0)),
            pl.BlockSpec((1, C), lambda i: (0, 0)),
            pl.BlockSpec((1, C), lambda i: (0, 0)),
            pl.BlockSpec((1, C), lambda i: (0, 0)),
            pl.BlockSpec((1, C), lambda i: (0, 0)),
        ],
        out_specs=pl.BlockSpec((_BLK, C), lambda i: (i, 0)),
        out_shape=jax.ShapeDtypeStruct((N_PAD, C), jnp.float32),
    )(x_padded, W, b, gamma, beta, x_pad)


_mesh = plsc.VectorSubcoreMesh(core_axis_name="c", subcore_axis_name="s")


@functools.partial(
    pl.kernel,
    out_type=jax.ShapeDtypeStruct((N_PAD, C), jnp.float32),
    mesh=_mesh,
    scratch_types=[
        pltpu.VMEM((N_PAD,), jnp.float32),
        pltpu.VMEM((N_PAD,), jnp.float32),
        pltpu.VMEM((N_PAD,), jnp.float32),
        pltpu.VMEM((PWMAX * K,), jnp.int32),
        pltpu.VMEM((PWMAX * K,), jnp.int32),
        pltpu.VMEM((PWMAX,), jnp.float32),
        pltpu.VMEM((PWMAX,), jnp.float32),
        pltpu.VMEM((PWMAX,), jnp.float32),
        pltpu.VMEM((PWMAX * K,), jnp.float32),
        pltpu.VMEM((2, G * K, C), jnp.float32),
        pltpu.VMEM((G, C), jnp.float32),
        pltpu.SemaphoreType.DMA,
        pltpu.SemaphoreType.DMA,
    ],
    compiler_params=pltpu.CompilerParams(needs_layout_passes=False),
)
def _sc_pool(table_hbm, idx_hbm, idxt_hbm, px_hbm, py_hbm, pz_hbm,
             qx_hbm, qy_hbm, qz_hbm, out_hbm,
             px_v, py_v, pz_v, idx_v, idxt_v, qx_v, qy_v, qz_v,
             w_v, rows_v, out_v, sem0, sem1):
    cid = lax.axis_index("c")
    sid = lax.axis_index("s")
    is_a = cid == 1
    pw = jnp.where(is_a, PW0, PW1)
    pbase = jnp.where(is_a, sid * PW0, A_PTS + sid * PW1)
    ss = jnp.where(is_a, PW0 // 16, PW1 // 16)
    ngrp = jnp.where(is_a, PW0 // G, PW1 // G)

    pltpu.sync_copy(px_hbm, px_v)
    pltpu.sync_copy(py_hbm, py_v)
    pltpu.sync_copy(pz_hbm, pz_v)
    pltpu.sync_copy(idx_hbm.at[pl.ds(pbase * K, PWMAX * K)], idx_v)
    pltpu.sync_copy(idxt_hbm.at[pl.ds(pbase * K, PWMAX * K)], idxt_v)
    pltpu.sync_copy(qx_hbm.at[pl.ds(pbase, PWMAX)], qx_v)
    pltpu.sync_copy(qy_hbm.at[pl.ds(pbase, PWMAX)], qy_v)
    pltpu.sync_copy(qz_hbm.at[pl.ds(pbase, PWMAX)], qz_v)

    def _start(g, buf, sem):
        idx_slice = idx_v.at[pl.ds(g * (G * K), G * K)]
        pltpu.async_copy(table_hbm.at[idx_slice], rows_v.at[buf], sem)

    def _wait(g, buf, sem):
        pltpu.make_async_copy(
            table_hbm.at[idx_v.at[pl.ds(g * (G * K), G * K)]],
            rows_v.at[buf], sem).wait()

    _start(0, 0, sem0)
    _start(1, 1, sem1)

    lanes_k = lax.iota(jnp.int32, 16) * K

    def _weights_body(s, carry):
        soff = pl.multiple_of(s * 16, 16)
        qx = qx_v[pl.ds(soff, 16)]
        qy = qy_v[pl.ds(soff, 16)]
        qz = qz_v[pl.ds(soff, 16)]
        den = jnp.zeros((16,), jnp.float32)
        wbase = s * (16 * K)
        for j in range(K):
            ivec = idxt_v[pl.ds(pl.multiple_of(j * pw + s * 16, 16), 16)]
            dx = qx - plsc.load_gather(px_v, [ivec])
            dy = qy - plsc.load_gather(py_v, [ivec])
            dz = qz - plsc.load_gather(pz_v, [ivec])
            d = dx * dx + dy * dy + dz * dz
            w = jnp.maximum(jnp.exp(d * _WSCALE), 0.001)
            den = den + w
            plsc.store_scatter(w_v, [wbase + lanes_k + j], w)
        inv = 1.0 / den
        for i in range(16):
            woff = pl.multiple_of(wbase + i * K, 16)
            w_v[pl.ds(woff, 16)] = w_v[pl.ds(woff, 16)] * inv[i]
            w_v[pl.ds(woff + 16, 16)] = w_v[pl.ds(woff + 16, 16)] * inv[i]
        return carry

    lax.fori_loop(0, ss, _weights_body, 0)

    def _group(g, buf):
        for i in range(G):
            woff = pl.multiple_of((g * G + i) * K, 16)
            wv0 = w_v[pl.ds(woff, 16)]
            wv1 = w_v[pl.ds(woff + 16, 16)]
            acc = [jnp.zeros((16,), jnp.float32) for _ in range(NV)]
            for j in range(K):
                wj = wv0[j] if j < 16 else wv1[j - 16]
                row = i * K + j
                for c in range(NV):
                    acc[c] = acc[c] + wj * rows_v[buf, row, pl.ds(c * 16, 16)]
            for c in range(NV):
                out_v[i, pl.ds(c * 16, 16)] = acc[c]
        pltpu.sync_copy(out_v, out_hbm.at[pl.ds(pbase + g * G, G)])

    def _pool_body(gp, carry):
        g = 2 * gp
        _wait(g, 0, sem0)
        _group(g, 0)

        @pl.when(g + 2 < ngrp)
        def _():
            _start(g + 2, 0, sem0)

        _wait(g + 1, 1, sem1)
        _group(g + 1, 1)

        @pl.when(g + 3 < ngrp)
        def _():
            _start(g + 3, 1, sem1)

        return carry

    lax.fori_loop(0, ngrp // 2, _pool_body, 0)


def kernel(x, x_pad, idx, neighbor_pts, query_pts, W, b, gamma, beta):
    x_padded = jnp.zeros((N_PAD, C), jnp.float32).at[:N].set(x)
    table = _dense(x_padded, W, b.reshape(1, C), gamma.reshape(1, C),
                   beta.reshape(1, C), x_pad)

    pts = jnp.full((N_PAD, 3), 1e6, jnp.float32).at[:N].set(neighbor_pts)
    q = jnp.zeros((N_PAD, 3), jnp.float32).at[:N].set(query_pts)
    idx_pad = jnp.zeros((N_PAD, K), jnp.int32).at[:N].set(idx.astype(jnp.int32))
    idx_flat = idx_pad.reshape(-1)
    idxt_a = idx_pad[:A_PTS].reshape(NS, PW0, K).transpose(0, 2, 1)
    idxt_b = idx_pad[A_PTS:].reshape(NS, PW1, K).transpose(0, 2, 1)
    idxt_flat = jnp.concatenate([idxt_a.reshape(-1), idxt_b.reshape(-1)])

    out = _sc_pool(table, idx_flat, idxt_flat,
                   pts[:, 0].copy(), pts[:, 1].copy(), pts[:, 2].copy(),
                   q[:, 0].copy(), q[:, 1].copy(), q[:, 2].copy())
    return out[:N]


# speedup vs baseline: 2.4126x; 2.2693x over previous
import functools

import jax
import jax.numpy as jnp
from jax import lax
from jax.experimental import pallas as pl
from jax.experimental.pallas import tpu as pltpu
from jax.experimental.pallas import tpu_sc as plsc

N = 10000
K = 32
C = 128
R = 0.1
NV = C // 16
CW = C // 2
RW = 128

N_PAD = 10240
NC = 2
NS = 16
NW = NC * NS
PW = N_PAD // NW
NIT = PW // 8
NSG = PW // 2
TROWS = N_PAD // NS

_SIGMA = R * 0.3
_WSCALE = -1.0 / (2.0 * _SIGMA * _SIGMA + 1e-9)

_BLK = 1280


def _dense_body(x_ref, w_ref, b_ref, g_ref, bt_ref, xpad_ref, o_ref):
    i = pl.program_id(0)
    h = lax.dot_general(x_ref[...], w_ref[...], (((1,), (1,)), ((), ())),
                        preferred_element_type=jnp.float32)
    h = h + b_ref[...]
    mu = jnp.mean(h, axis=-1, keepdims=True)
    var = jnp.mean((h - mu) ** 2, axis=-1, keepdims=True)
    h = (h - mu) * lax.rsqrt(var + 1e-5) * g_ref[...] + bt_ref[...]
    h = jnp.where(h >= 0, h, 0.1 * h)
    rows = i * _BLK + lax.broadcasted_iota(jnp.int32, (_BLK, 1), 0)
    o_ref[...] = jnp.where(rows >= N, xpad_ref[...], h).astype(jnp.bfloat16)


def _dense(x_padded, W, b, gamma, beta, x_pad):
    return pl.pallas_call(
        _dense_body,
        grid=(N_PAD // _BLK,),
        in_specs=[
            pl.BlockSpec((_BLK, C), lambda i: (i, 0)),
            pl.BlockSpec((C, C), lambda i: (0, 0)),
            pl.BlockSpec((1, C), lambda i: (0, 0)),
            pl.BlockSpec((1, C), lambda i: (0, 0)),
            pl.BlockSpec((1, C), lambda i: (0, 0)),
            pl.BlockSpec((1, C), lambda i: (0, 0)),
        ],
        out_specs=pl.BlockSpec((_BLK, C), lambda i: (i, 0)),
        out_shape=jax.ShapeDtypeStruct((N_PAD, C), jnp.bfloat16),
    )(x_padded, W, b, gamma, beta, x_pad)


_mesh = plsc.VectorSubcoreMesh(core_axis_name="c", subcore_axis_name="s")


@functools.partial(
    pl.kernel,
    out_type=jax.ShapeDtypeStruct((N_PAD, C), jnp.float32),
    mesh=_mesh,
    scratch_types=[
        pltpu.VMEM_SHARED((N_PAD, RW), jnp.int32),
        pltpu.VMEM((PW * K,), jnp.int32),
        pltpu.VMEM((PW,), jnp.float32),
        pltpu.VMEM((PW,), jnp.float32),
        pltpu.VMEM((PW,), jnp.float32),
        pltpu.VMEM((2, 2 * K, RW), jnp.int32),
        pltpu.VMEM((2, C), jnp.float32),
        pltpu.SemaphoreType.DMA,
        pltpu.SemaphoreType.DMA,
        pltpu.SemaphoreType.DMA,
    ],
    compiler_params=pltpu.CompilerParams(needs_layout_passes=False),
)
def _sc_pool(tbl_hbm, idx_hbm, qx_hbm, qy_hbm, qz_hbm, out_hbm,
             tab_s, idx_v, qx_v, qy_v, qz_v,
             rows_v, out_v, sem0, sem1, semt):
    wid = lax.axis_index("s") * NC + lax.axis_index("c")
    sid = lax.axis_index("s")
    pbase = wid * PW

    pltpu.async_copy(tbl_hbm.at[pl.ds(sid * TROWS, TROWS)],
                     tab_s.at[pl.ds(sid * TROWS, TROWS)], semt)

    pltpu.sync_copy(idx_hbm.at[pl.ds(pbase * K, PW * K)], idx_v)
    pltpu.sync_copy(qx_hbm.at[pl.ds(pbase, PW)], qx_v)
    pltpu.sync_copy(qy_hbm.at[pl.ds(pbase, PW)], qy_v)
    pltpu.sync_copy(qz_hbm.at[pl.ds(pbase, PW)], qz_v)

    pltpu.make_async_copy(tbl_hbm.at[pl.ds(sid * TROWS, TROWS)],
                          tab_s.at[pl.ds(sid * TROWS, TROWS)], semt).wait()
    plsc.subcore_barrier()

    sems = (sem0, sem1)

    def _start(sg, buf, sem):
        idx_slice = idx_v.at[pl.ds(sg * (2 * K), 2 * K)]
        pltpu.async_copy(tab_s.at[idx_slice], rows_v.at[buf], sem)

    def _wait(sg, buf, sem):
        pltpu.make_async_copy(
            tab_s.at[idx_v.at[pl.ds(sg * (2 * K), 2 * K)]],
            rows_v.at[buf], sem).wait()

    _start(0, 0, sem0)
    _start(1, 1, sem1)

    iota16 = lax.iota(jnp.int32, 16)

    def _point(buf, i, qx, qy, qz):
        buf_i = jnp.full((16,), buf, jnp.int32)
        r0 = i * K + iota16
        r1 = r0 + 16
        c64 = jnp.full((16,), CW, jnp.int32)
        c65 = jnp.full((16,), CW + 1, jnp.int32)
        c66 = jnp.full((16,), CW + 2, jnp.int32)
        px0 = plsc.bitcast(plsc.load_gather(rows_v, [buf_i, r0, c64]),
                           jnp.float32)
        py0 = plsc.bitcast(plsc.load_gather(rows_v, [buf_i, r0, c65]),
                           jnp.float32)
        pz0 = plsc.bitcast(plsc.load_gather(rows_v, [buf_i, r0, c66]),
                           jnp.float32)
        px1 = plsc.bitcast(plsc.load_gather(rows_v, [buf_i, r1, c64]),
                           jnp.float32)
        py1 = plsc.bitcast(plsc.load_gather(rows_v, [buf_i, r1, c65]),
                           jnp.float32)
        pz1 = plsc.bitcast(plsc.load_gather(rows_v, [buf_i, r1, c66]),
                           jnp.float32)
        dx0, dy0, dz0 = qx - px0, qy - py0, qz - pz0
        dx1, dy1, dz1 = qx - px1, qy - py1, qz - pz1
        d0 = dx0 * dx0 + dy0 * dy0 + dz0 * dz0
        d1 = dx1 * dx1 + dy1 * dy1 + dz1 * dz1
        w0 = jnp.maximum(jnp.exp(d0 * _WSCALE), 0.001)
        w1 = jnp.maximum(jnp.exp(d1 * _WSCALE), 0.001)
        den = jnp.broadcast_to(jnp.sum(w0 + w1), (16,))
        inv = 1.0 / den
        wn0 = w0 * inv
        wn1 = w1 * inv

        acc = [jnp.zeros((16,), jnp.float32) for _ in range(NV)]
        for j in range(K):
            wj = wn0[j] if j < 16 else wn1[j - 16]
            row = i * K + j
            for blk in range(C // 32):
                v = plsc.bitcast(rows_v[buf, row, pl.ds(blk * 16, 16)],
                                 jnp.bfloat16)
                a, b = plsc.unpack(v, format=plsc.PackFormat.INTERLEAVED)
                acc[2 * blk] = acc[2 * blk] + wj * a
                acc[2 * blk + 1] = acc[2 * blk + 1] + wj * b
        for c in range(NV):
            out_v[i, pl.ds(c * 16, 16)] = acc[c]

    def _iter_body(it, carry):
        qoff = pl.multiple_of(it * 8, 8)
        qxv = qx_v[pl.ds(qoff, 16)]
        qyv = qy_v[pl.ds(qoff, 16)]
        qzv = qz_v[pl.ds(qoff, 16)]
        for t in range(4):
            sg = it * 4 + t
            buf = t % 2
            _wait(sg, buf, sems[buf])
            for i in range(2):
                lane = 2 * t + i
                _point(buf, i, qxv[lane], qyv[lane], qzv[lane])
            pltpu.sync_copy(out_v, out_hbm.at[pl.ds(pbase + sg * 2, 2)])

            @pl.when(sg + 2 < NSG)
            def _():
                _start(sg + 2, buf, sems[buf])

        return carry

    lax.fori_loop(0, NIT, _iter_body, 0)


_PERM = [0] * C
for _blk in range(C // 32):
    for _t in range(16):
        _PERM[_blk * 32 + 2 * _t] = _blk * 32 + _t
        _PERM[_blk * 32 + 2 * _t + 1] = _blk * 32 + 16 + _t
_PERM = tuple(_PERM)


def kernel(x, x_pad, idx, neighbor_pts, query_pts, W, b, gamma, beta):
    perm = jnp.asarray(_PERM, jnp.int32)
    x_padded = jnp.zeros((N_PAD, C), jnp.float32).at[:N].set(x)
    table = _dense(x_padded, W[perm, :], b[perm].reshape(1, C),
                   gamma[perm].reshape(1, C), beta[perm].reshape(1, C),
                   x_pad[:, perm])
    feat_i32 = lax.bitcast_convert_type(
        table.reshape(N_PAD, CW, 2), jnp.int32)

    pts = jnp.full((N_PAD, 3), 1e6, jnp.float32).at[:N].set(neighbor_pts)
    pts_i32 = lax.bitcast_convert_type(pts, jnp.int32)
    fused = jnp.concatenate(
        [feat_i32, pts_i32,
         jnp.zeros((N_PAD, RW - CW - 3), jnp.int32)], axis=1)

    q = jnp.zeros((N_PAD, 3), jnp.float32).at[:N].set(query_pts)
    idx_pad = jnp.zeros((N_PAD, K), jnp.int32).at[:N].set(idx.astype(jnp.int32))
    idx_flat = idx_pad.reshape(-1)

    out = _sc_pool(fused, idx_flat,
                   q[:, 0].copy(), q[:, 1].copy(), q[:, 2].copy())
    return out[:N]
